# baseline (speedup 1.0000x reference)
import dataclasses
import functools

import jax
import jax.numpy as jnp
from jax import lax
from jax.experimental import pallas as pl
from jax.experimental.pallas import tpu as pltpu
from jax.experimental.pallas import tpu_sc as plsc

N = 10000
D = 128
P = 10112
E = 320000
CH = 64
EPAD = 327680
NCHUNK = EPAD // CH
NC, NS = 2, 16
NW = NC * NS
CPT = NCHUNK // NW
SLOWC = 0
CPTS = 80
CPTF = 2 * CPT - CPTS
GPT = EPAD // NW // 16
RPT = P // NS
BN = 2000

_mesh = plsc.VectorSubcoreMesh(core_axis_name="c", subcore_axis_name="s")

_cp = pltpu.CompilerParams()
if "needs_layout_passes" in pltpu.CompilerParams.__dataclass_fields__:
    _cp = dataclasses.replace(_cp, needs_layout_passes=False)


def _rmw_min6(acc_ref, fidxs, vals6):
    def step(masks):
        outs = []
        for fidx, vals, m in zip(fidxs, vals6, masks):
            cur = plsc.load_gather(acc_ref, [fidx], mask=m)
            new = jnp.minimum(cur, vals)
            plsc.store_scatter(acc_ref, [fidx], new, mask=m)
            chk = plsc.load_gather(acc_ref, [fidx], mask=m)
            outs.append(m & (chk > new))
        return outs

    ones = jnp.ones((16,), jnp.bool_)
    pend = step([ones] * 6)

    def cond(ms):
        return jnp.any(ms[0] | ms[1] | ms[2] | ms[3] | ms[4] | ms[5])

    lax.while_loop(cond, step, pend)


SLT = 61440
SL = SLT // NW


CH1 = 128
CPT1 = EPAD // NW // CH1


@functools.partial(
    pl.kernel,
    mesh=_mesh,
    out_type=jax.ShapeDtypeStruct((NW, SLT), jnp.float32),
    scratch_types=[
        pltpu.VMEM((2, CPT1, CH1), jnp.int32),
        pltpu.VMEM((3 * P,), jnp.float32),
        pltpu.VMEM((SLT,), jnp.float32),
    ],
    compiler_params=_cp,
)
def _k1(rowcol_hbm, post_hbm, out_hbm, idx_v, pos_v, acc_v):
    c = lax.axis_index("c")
    s = lax.axis_index("s")
    w = s * NC + c
    pltpu.sync_copy(rowcol_hbm.at[:, pl.ds(w * CPT1, CPT1), :], idx_v)
    pltpu.sync_copy(post_hbm, pos_v)
    inf16 = jnp.full((16,), jnp.inf, jnp.float32)

    @pl.loop(0, SLT // 16)
    def _(i):
        acc_v[pl.ds(i * 16, 16)] = inf16

    @pl.loop(0, GPT)
    def _(g):
        r = g // (CH1 // 16)
        cc = (g % (CH1 // 16)) * 16
        row16 = idx_v[0, r, pl.ds(cc, 16)]
        col16 = idx_v[1, r, pl.ds(cc, 16)]
        comp = []
        for k in range(3):
            comp.append(plsc.load_gather(pos_v, [row16 + k * P]))
        base = col16 * 6
        fidxs = [base + k for k in range(6)]
        vals6 = [comp[0], comp[1], comp[2], -comp[0], -comp[1], -comp[2]]
        _rmw_min6(acc_v, fidxs, vals6)

    pltpu.sync_copy(acc_v, out_hbm.at[w])


@functools.partial(
    pl.kernel,
    mesh=_mesh,
    out_type=jax.ShapeDtypeStruct((NW, SL), jnp.float32),
    scratch_types=[
        pltpu.VMEM((NW * (SL // 128), 128), jnp.float32),
        pltpu.VMEM((SL,), jnp.float32),
        pltpu.SemaphoreType.DMA,
    ],
    compiler_params=_cp,
)
def _k1b(parts_hbm, out_hbm, buf, acc, sem):
    c = lax.axis_index("c")
    s = lax.axis_index("s")
    w = s * NC + c
    nr = SL // 128
    for k in range(NW):
        pltpu.make_async_copy(parts_hbm.at[k, w],
                              buf.at[pl.ds(k * nr, nr)], sem).start()
    for k in range(NW):
        pltpu.make_async_copy(parts_hbm.at[k, w],
                              buf.at[pl.ds(k * nr, nr)], sem).wait()

    @pl.loop(0, SL // 16)
    def _(g):
        gg = g // 8
        o = (g % 8) * 16
        m = buf[gg, pl.ds(o, 16)]
        for k in range(1, NW):
            m = jnp.minimum(m, buf[k * nr + gg, pl.ds(o, 16)])
        acc[pl.ds(g * 16, 16)] = m

    pltpu.sync_copy(acc, out_hbm.at[w])


@functools.partial(
    pl.kernel,
    mesh=_mesh,
    out_type=[
        jax.ShapeDtypeStruct((NC, P, D), jnp.float32),
        jax.ShapeDtypeStruct((NC * P,), jnp.float32),
    ],
    scratch_types=[
        pltpu.VMEM((2, 2, 8, CH), jnp.int32),
        pltpu.VMEM((2, CH, D), jnp.float32),
        pltpu.VMEM((2, CH, D), jnp.float32),
        pltpu.VMEM((CH, D), jnp.float32),
        pltpu.VMEM((RPT + 8,), jnp.float32),
        pltpu.VMEM((CH,), jnp.float32),
        pltpu.VMEM((D,), jnp.float32),
        pltpu.SemaphoreType.DMA((2,)),
        pltpu.SemaphoreType.DMA((2,)),
        pltpu.SemaphoreType.DMA((2,)),
        pltpu.SemaphoreType.DMA,
        pltpu.SemaphoreType.DMA,
        pltpu.VMEM_SHARED((P, D), jnp.float32),
        pltpu.VMEM_SHARED((P,), jnp.float32),
    ],
    compiler_params=_cp,
)
def _k3(p_hbm, rowcol_hbm, b1_hbm, s_out, d_out,
        idx_v, bufA, bufB, hbuf, zb, ones_v, b1_v, semI, semA, semB,
        semS, semD, s_acc, deg_acc):
    c = lax.axis_index("c")
    s = lax.axis_index("s")
    cpt = jnp.where(c == SLOWC, CPTS, CPTF)
    cbase = jnp.where(c == SLOWC, s * CPTS,
                      16 * CPTS + s * CPTF)
    pltpu.sync_copy(b1_hbm, b1_v)

    zero16 = jnp.zeros((16,), jnp.float32)
    one16 = jnp.ones((16,), jnp.float32)

    @pl.loop(0, CH * D // 16)
    def _(i):
        bufA[0, i // (D // 16), pl.ds((i % (D // 16)) * 16, 16)] = zero16

    @pl.loop(0, (RPT + 8) // 16)
    def _(i):
        zb[pl.ds(i * 16, 16)] = zero16

    @pl.loop(0, CH // 16)
    def _(i):
        ones_v[pl.ds(i * 16, 16)] = one16

    base = s * RPT
    for j in range(RPT // CH):
        pltpu.sync_copy(bufA.at[0], s_acc.at[pl.ds(base + j * CH, CH)])
    rem = RPT - (RPT // CH) * CH
    pltpu.sync_copy(bufA.at[0, pl.ds(0, rem)],
                    s_acc.at[pl.ds(base + RPT - rem, rem)])
    pltpu.sync_copy(zb.at[pl.ds(0, RPT)], deg_acc.at[pl.ds(base, RPT)])
    plsc.subcore_barrier()

    b1s = [b1_v[pl.ds(q * 16, 16)] for q in range(D // 16)]

    def idx_fetch(bs, slot):
        pltpu.make_async_copy(rowcol_hbm.at[:, pl.ds(cbase + bs, 8), :],
                              idx_v.at[slot], semI.at[slot]).start()

    def idx_wait(bs, slot):
        pltpu.make_async_copy(rowcol_hbm.at[:, pl.ds(cbase + bs, 8), :],
                              idx_v.at[slot], semI.at[slot]).wait()

    def gathers(slot, r, b):
        pltpu.make_async_copy(p_hbm.at[idx_v.at[slot, 0, r]], bufA.at[b],
                              semA.at[b]).start()
        pltpu.make_async_copy(p_hbm.at[idx_v.at[slot, 1, r]], bufB.at[b],
                              semB.at[b]).start()

    def wait_gathers(slot, r, b):
        pltpu.make_async_copy(p_hbm.at[idx_v.at[slot, 0, r]], bufA.at[b],
                              semA.at[b]).wait()
        pltpu.make_async_copy(p_hbm.at[idx_v.at[slot, 1, r]], bufB.at[b],
                              semB.at[b]).wait()

    idx_fetch(0, 0)
    idx_fetch(8, 1)
    idx_wait(0, 0)
    gathers(0, 0, 0)
    gathers(0, 1, 1)

    def sscatter(slot, r):
        return pltpu.make_async_copy(hbuf, s_acc.at[idx_v.at[slot, 1, r]],
                                     semS)

    def dscatter(slot, r):
        return pltpu.make_async_copy(ones_v, deg_acc.at[idx_v.at[slot, 1, r]],
                                     semD)

    @pl.loop(0, cpt, step=16)
    def _(j0):
        for u in range(16):
            j = j0 + u
            b = u % 2
            slot = u // 8
            r = u % 8
            wait_gathers(slot, r, b)

            @pl.when(j > 0)
            def _():
                sscatter(slot, r).wait()

            @pl.loop(0, CH)
            def _(rr):
                for q in range(D // 16):
                    sl = pl.ds(q * 16, 16)
                    hbuf[rr, sl] = jnp.maximum(
                        bufA[b, rr, sl] - bufB[b, rr, sl] + b1s[q], 0.0)

            sscatter(slot, r).start(add=True)
            dscatter(slot, r).start(add=True)

            if u == 7:
                @pl.when(j0 + 16 < cpt)
                def _():
                    idx_fetch(j0 + 16, 0)
            if u == 15:
                @pl.when(j0 + 24 < cpt)
                def _():
                    idx_fetch(j0 + 24, 1)

            s2 = ((u + 2) // 8) % 2
            r2 = (u + 2) % 8

            @pl.when(j + 2 < cpt)
            def _():
                if u == 6:
                    idx_wait(j0 + 8, 1)
                if u == 14:
                    idx_wait(j0 + 16, 0)
                gathers(s2, r2, b)

    sscatter(1, 7).wait()

    @pl.loop(0, cpt)
    def _(j):
        dscatter(0, 0).wait()

    plsc.subcore_barrier()
    pltpu.sync_copy(s_acc.at[pl.ds(base, RPT)], s_out.at[c, pl.ds(base, RPT)])
    pltpu.sync_copy(deg_acc.at[pl.ds(base, RPT)], zb.at[pl.ds(0, RPT)])
    pltpu.sync_copy(zb.at[pl.ds(0, RPT)], d_out.at[pl.ds(c * P + base, RPT)])


def _k2_body(rm_ref, x_ref, pos_ref, w1mm_ref, w1p_ref, w1x_ref, o_ref):
    rm = rm_ref[0:N, :]
    lane = lax.broadcasted_iota(jnp.int32, rm.shape, 1)
    pp = jnp.concatenate([pos_ref[...], pos_ref[...]], axis=1)
    tm = jnp.where(lane < 3, rm, -rm) - pp
    tm = jnp.where(jnp.isfinite(rm), tm, 0.0)
    o_ref[...] = (
        jnp.dot(tm, w1mm_ref[...], preferred_element_type=jnp.float32)
        + jnp.dot(pos_ref[...], w1p_ref[...], preferred_element_type=jnp.float32)
        + jnp.dot(x_ref[...], w1x_ref[...], preferred_element_type=jnp.float32)
    )


_k2 = pl.pallas_call(
    _k2_body,
    out_shape=jax.ShapeDtypeStruct((N, D), jnp.float32),
)


def _k4_body(s_ref, d_ref, w2_ref, b2_ref, o_ref):
    sv = s_ref[0, 0:N, :] + s_ref[1, 0:N, :]
    deg = (d_ref[0, 0:N] + d_ref[1, 0:N]).reshape(-1, 1)
    o_ref[...] = (jnp.dot(sv, w2_ref[...], preferred_element_type=jnp.float32)
                  + deg * b2_ref[...])


_k4 = pl.pallas_call(
    _k4_body,
    out_shape=jax.ShapeDtypeStruct((N, D), jnp.float32),
)


def kernel(x, pos, edges, W1, b1, W2, b2):
    row = edges[0].astype(jnp.int32)
    col = edges[1].astype(jnp.int32)
    rowp = jnp.concatenate([row, jnp.zeros((EPAD - E,), jnp.int32)])
    colp = jnp.concatenate([col, N + jnp.arange(EPAD - E, dtype=jnp.int32) % (P - N)])
    rc = jnp.stack([rowp, colp])
    rowcol1 = rc.reshape(2, NCHUNK // 2, CH1)
    rowcol3 = rc.reshape(2, NCHUNK, CH)
    post = jnp.pad(pos.T, ((0, 0), (0, P - N))).reshape(3 * P)

    part = _k1(rowcol1, post)
    rm = _k1b(part.reshape(NW, NW, SL // 128, 128))
    rm = rm.reshape(SLT)[:6 * P].reshape(P, 6)
    p = _k2(rm, x, pos, W1[0:6], W1[6:9], W1[9:137])
    p_pad = jnp.pad(p, ((0, P - N), (0, 0)))
    s2, d2 = _k3(p_pad, rowcol3, b1)
    out = _k4(s2, d2.reshape(NC, P), W2, b2.reshape(1, D))
    return out


# speedup vs baseline: 1.0456x; 1.0456x over previous
import dataclasses
import functools

import jax
import jax.numpy as jnp
from jax import lax
from jax.experimental import pallas as pl
from jax.experimental.pallas import tpu as pltpu
from jax.experimental.pallas import tpu_sc as plsc

N = 10000
D = 128
P = 10112
E = 320000
CH = 64
EPAD = 327680
NCHUNK = EPAD // CH
NC, NS = 2, 16
NW = NC * NS
CPT = NCHUNK // NW
SLOWC = 1
CPTS = 80
CPTF = 2 * CPT - CPTS
GPT = EPAD // NW // 16
RPT = P // NS
BN = 2000

_mesh = plsc.VectorSubcoreMesh(core_axis_name="c", subcore_axis_name="s")

_cp = pltpu.CompilerParams()
if "needs_layout_passes" in pltpu.CompilerParams.__dataclass_fields__:
    _cp = dataclasses.replace(_cp, needs_layout_passes=False)


def _rmw_min6(acc_ref, fidxs, vals6):
    def step(masks):
        outs = []
        for fidx, vals, m in zip(fidxs, vals6, masks):
            cur = plsc.load_gather(acc_ref, [fidx], mask=m)
            new = jnp.minimum(cur, vals)
            plsc.store_scatter(acc_ref, [fidx], new, mask=m)
            chk = plsc.load_gather(acc_ref, [fidx], mask=m)
            outs.append(m & (chk > new))
        return outs

    ones = jnp.ones((16,), jnp.bool_)
    pend = step([ones] * 6)

    def cond(ms):
        return jnp.any(ms[0] | ms[1] | ms[2] | ms[3] | ms[4] | ms[5])

    lax.while_loop(cond, step, pend)


SLT = 61440
SL = SLT // NW


CH1 = 128
CPT1 = EPAD // NW // CH1


@functools.partial(
    pl.kernel,
    mesh=_mesh,
    out_type=jax.ShapeDtypeStruct((NW, SLT), jnp.float32),
    scratch_types=[
        pltpu.VMEM((2, CPT1, CH1), jnp.int32),
        pltpu.VMEM((3 * P,), jnp.float32),
        pltpu.VMEM((SLT,), jnp.float32),
    ],
    compiler_params=_cp,
)
def _k1(rowcol_hbm, post_hbm, out_hbm, idx_v, pos_v, acc_v):
    c = lax.axis_index("c")
    s = lax.axis_index("s")
    w = s * NC + c
    pltpu.sync_copy(rowcol_hbm.at[:, pl.ds(w * CPT1, CPT1), :], idx_v)
    pltpu.sync_copy(post_hbm, pos_v)
    inf16 = jnp.full((16,), jnp.inf, jnp.float32)

    @pl.loop(0, SLT // 16)
    def _(i):
        acc_v[pl.ds(i * 16, 16)] = inf16

    @pl.loop(0, GPT)
    def _(g):
        r = g // (CH1 // 16)
        cc = (g % (CH1 // 16)) * 16
        row16 = idx_v[0, r, pl.ds(cc, 16)]
        col16 = idx_v[1, r, pl.ds(cc, 16)]
        comp = []
        for k in range(3):
            comp.append(plsc.load_gather(pos_v, [row16 + k * P]))
        base = col16 * 6
        fidxs = [base + k for k in range(6)]
        vals6 = [comp[0], comp[1], comp[2], -comp[0], -comp[1], -comp[2]]
        _rmw_min6(acc_v, fidxs, vals6)

    pltpu.sync_copy(acc_v, out_hbm.at[w])


@functools.partial(
    pl.kernel,
    mesh=_mesh,
    out_type=jax.ShapeDtypeStruct((NW, SL), jnp.float32),
    scratch_types=[
        pltpu.VMEM((NW * (SL // 128), 128), jnp.float32),
        pltpu.VMEM((SL,), jnp.float32),
        pltpu.SemaphoreType.DMA,
    ],
    compiler_params=_cp,
)
def _k1b(parts_hbm, out_hbm, buf, acc, sem):
    c = lax.axis_index("c")
    s = lax.axis_index("s")
    w = s * NC + c
    nr = SL // 128
    for k in range(NW):
        pltpu.make_async_copy(parts_hbm.at[k, w],
                              buf.at[pl.ds(k * nr, nr)], sem).start()
    for k in range(NW):
        pltpu.make_async_copy(parts_hbm.at[k, w],
                              buf.at[pl.ds(k * nr, nr)], sem).wait()

    @pl.loop(0, SL // 16)
    def _(g):
        gg = g // 8
        o = (g % 8) * 16
        m = buf[gg, pl.ds(o, 16)]
        for k in range(1, NW):
            m = jnp.minimum(m, buf[k * nr + gg, pl.ds(o, 16)])
        acc[pl.ds(g * 16, 16)] = m

    pltpu.sync_copy(acc, out_hbm.at[w])


@functools.partial(
    pl.kernel,
    mesh=_mesh,
    out_type=[
        jax.ShapeDtypeStruct((NC, P, D), jnp.float32),
        jax.ShapeDtypeStruct((NC * P,), jnp.float32),
    ],
    scratch_types=[
        pltpu.VMEM((2, 2, 8, CH), jnp.int32),
        pltpu.VMEM((2, CH, D), jnp.float32),
        pltpu.VMEM((2, CH, D), jnp.float32),
        pltpu.VMEM((CH, D), jnp.float32),
        pltpu.VMEM((RPT + 8,), jnp.float32),
        pltpu.VMEM((CH,), jnp.float32),
        pltpu.VMEM((D,), jnp.float32),
        pltpu.SemaphoreType.DMA((2,)),
        pltpu.SemaphoreType.DMA((2,)),
        pltpu.SemaphoreType.DMA((2,)),
        pltpu.SemaphoreType.DMA,
        pltpu.SemaphoreType.DMA,
        pltpu.VMEM_SHARED((P, D), jnp.float32),
        pltpu.VMEM_SHARED((P,), jnp.float32),
    ],
    compiler_params=_cp,
)
def _k3(p_hbm, rowcol_hbm, b1_hbm, s_out, d_out,
        idx_v, bufA, bufB, hbuf, zb, ones_v, b1_v, semI, semA, semB,
        semS, semD, s_acc, deg_acc):
    c = lax.axis_index("c")
    s = lax.axis_index("s")
    cpt = jnp.where(c == SLOWC, CPTS, CPTF)
    cbase = jnp.where(c == SLOWC, s * CPTS,
                      16 * CPTS + s * CPTF)
    pltpu.sync_copy(b1_hbm, b1_v)

    zero16 = jnp.zeros((16,), jnp.float32)
    one16 = jnp.ones((16,), jnp.float32)

    @pl.loop(0, CH * D // 16)
    def _(i):
        bufA[0, i // (D // 16), pl.ds((i % (D // 16)) * 16, 16)] = zero16

    @pl.loop(0, (RPT + 8) // 16)
    def _(i):
        zb[pl.ds(i * 16, 16)] = zero16

    @pl.loop(0, CH // 16)
    def _(i):
        ones_v[pl.ds(i * 16, 16)] = one16

    base = s * RPT
    for j in range(RPT // CH):
        pltpu.sync_copy(bufA.at[0], s_acc.at[pl.ds(base + j * CH, CH)])
    rem = RPT - (RPT // CH) * CH
    pltpu.sync_copy(bufA.at[0, pl.ds(0, rem)],
                    s_acc.at[pl.ds(base + RPT - rem, rem)])
    pltpu.sync_copy(zb.at[pl.ds(0, RPT)], deg_acc.at[pl.ds(base, RPT)])
    plsc.subcore_barrier()

    b1s = [b1_v[pl.ds(q * 16, 16)] for q in range(D // 16)]

    def idx_fetch(bs, slot):
        pltpu.make_async_copy(rowcol_hbm.at[:, pl.ds(cbase + bs, 8), :],
                              idx_v.at[slot], semI.at[slot]).start()

    def idx_wait(bs, slot):
        pltpu.make_async_copy(rowcol_hbm.at[:, pl.ds(cbase + bs, 8), :],
                              idx_v.at[slot], semI.at[slot]).wait()

    def gathers(slot, r, b):
        pltpu.make_async_copy(p_hbm.at[idx_v.at[slot, 0, r]], bufA.at[b],
                              semA.at[b]).start()
        pltpu.make_async_copy(p_hbm.at[idx_v.at[slot, 1, r]], bufB.at[b],
                              semB.at[b]).start()

    def wait_gathers(slot, r, b):
        pltpu.make_async_copy(p_hbm.at[idx_v.at[slot, 0, r]], bufA.at[b],
                              semA.at[b]).wait()
        pltpu.make_async_copy(p_hbm.at[idx_v.at[slot, 1, r]], bufB.at[b],
                              semB.at[b]).wait()

    idx_fetch(0, 0)
    idx_fetch(8, 1)
    idx_wait(0, 0)
    gathers(0, 0, 0)
    gathers(0, 1, 1)

    def sscatter(slot, r):
        return pltpu.make_async_copy(hbuf, s_acc.at[idx_v.at[slot, 1, r]],
                                     semS)

    def dscatter(slot, r):
        return pltpu.make_async_copy(ones_v, deg_acc.at[idx_v.at[slot, 1, r]],
                                     semD)

    @pl.loop(0, cpt, step=16)
    def _(j0):
        for u in range(16):
            j = j0 + u
            b = u % 2
            slot = u // 8
            r = u % 8
            wait_gathers(slot, r, b)

            @pl.when(j > 0)
            def _():
                sscatter(slot, r).wait()

            @pl.loop(0, CH)
            def _(rr):
                for q in range(D // 16):
                    sl = pl.ds(q * 16, 16)
                    hbuf[rr, sl] = jnp.maximum(
                        bufA[b, rr, sl] - bufB[b, rr, sl] + b1s[q], 0.0)

            sscatter(slot, r).start(add=True)
            dscatter(slot, r).start(add=True)

            if u == 7:
                @pl.when(j0 + 16 < cpt)
                def _():
                    idx_fetch(j0 + 16, 0)
            if u == 15:
                @pl.when(j0 + 24 < cpt)
                def _():
                    idx_fetch(j0 + 24, 1)

            s2 = ((u + 2) // 8) % 2
            r2 = (u + 2) % 8

            @pl.when(j + 2 < cpt)
            def _():
                if u == 6:
                    idx_wait(j0 + 8, 1)
                if u == 14:
                    idx_wait(j0 + 16, 0)
                gathers(s2, r2, b)

    sscatter(1, 7).wait()

    @pl.loop(0, cpt)
    def _(j):
        dscatter(0, 0).wait()

    plsc.subcore_barrier()
    pltpu.sync_copy(s_acc.at[pl.ds(base, RPT)], s_out.at[c, pl.ds(base, RPT)])
    pltpu.sync_copy(deg_acc.at[pl.ds(base, RPT)], zb.at[pl.ds(0, RPT)])
    pltpu.sync_copy(zb.at[pl.ds(0, RPT)], d_out.at[pl.ds(c * P + base, RPT)])


def _k2_body(rm_ref, x_ref, pos_ref, w1mm_ref, w1p_ref, w1x_ref, o_ref):
    rm = rm_ref[0:N, :]
    lane = lax.broadcasted_iota(jnp.int32, rm.shape, 1)
    pp = jnp.concatenate([pos_ref[...], pos_ref[...]], axis=1)
    tm = jnp.where(lane < 3, rm, -rm) - pp
    tm = jnp.where(jnp.isfinite(rm), tm, 0.0)
    o_ref[...] = (
        jnp.dot(tm, w1mm_ref[...], preferred_element_type=jnp.float32)
        + jnp.dot(pos_ref[...], w1p_ref[...], preferred_element_type=jnp.float32)
        + jnp.dot(x_ref[...], w1x_ref[...], preferred_element_type=jnp.float32)
    )


_k2 = pl.pallas_call(
    _k2_body,
    out_shape=jax.ShapeDtypeStruct((N, D), jnp.float32),
)


def _k4_body(s_ref, d_ref, w2_ref, b2_ref, o_ref):
    sv = s_ref[0, 0:N, :] + s_ref[1, 0:N, :]
    deg = (d_ref[0, 0:N] + d_ref[1, 0:N]).reshape(-1, 1)
    o_ref[...] = (jnp.dot(sv, w2_ref[...], preferred_element_type=jnp.float32)
                  + deg * b2_ref[...])


_k4 = pl.pallas_call(
    _k4_body,
    out_shape=jax.ShapeDtypeStruct((N, D), jnp.float32),
)


def kernel(x, pos, edges, W1, b1, W2, b2):
    row = edges[0].astype(jnp.int32)
    col = edges[1].astype(jnp.int32)
    rowp = jnp.concatenate([row, jnp.zeros((EPAD - E,), jnp.int32)])
    colp = jnp.concatenate([col, N + jnp.arange(EPAD - E, dtype=jnp.int32) % (P - N)])
    rc = jnp.stack([rowp, colp])
    rowcol1 = rc.reshape(2, NCHUNK // 2, CH1)
    rowcol3 = rc.reshape(2, NCHUNK, CH)
    post = jnp.pad(pos.T, ((0, 0), (0, P - N))).reshape(3 * P)

    part = _k1(rowcol1, post)
    rm = _k1b(part.reshape(NW, NW, SL // 128, 128))
    rm = rm.reshape(SLT)[:6 * P].reshape(P, 6)
    p = _k2(rm, x, pos, W1[0:6], W1[6:9], W1[9:137])
    p_pad = jnp.pad(p, ((0, P - N), (0, 0)))
    s2, d2 = _k3(p_pad, rowcol3, b1)
    out = _k4(s2, d2.reshape(NC, P), W2, b2.reshape(1, D))
    return out
